# baseline (speedup 1.0000x reference)
import functools

import jax
import jax.numpy as jnp
from jax.experimental import pallas as pl
from jax.experimental.pallas import tpu as pltpu


_BLOCK_B = 256
_NSTREAMS = 4
_LN2 = 0.6931471805599453


def _body(y_ref, *refs, nsteps, scale):
    i = pl.program_id(0)
    ns = _NSTREAMS
    x_refs = refs[:ns]
    o_ref = refs[ns]
    acc_ref = refs[ns + 1]
    bb = _BLOCK_B

    part = None
    for s, x_ref in enumerate(x_refs):
        lg = jnp.log2(x_ref[...])
        yb = y_ref[pl.ds((s * nsteps + i) * bb, bb), :]
        cols = jax.lax.broadcasted_iota(jnp.int32, lg.shape, 1)
        m = jnp.where(cols == yb, 0.0, lg)
        p = jnp.sum(m.reshape(m.shape[0] // 8, 8, m.shape[1]), axis=0)
        part = p if part is None else part + p

    @pl.when(i == 0)
    def _():
        acc_ref[...] = jnp.zeros_like(acc_ref)

    acc_ref[...] += part

    @pl.when(i == nsteps - 1)
    def _():
        o_ref[...] = jnp.sum(acc_ref[...]).reshape(1, 1) * scale


def kernel(y_pred, y):
    B, C = y_pred.shape
    bb = _BLOCK_B
    ns = _NSTREAMS
    nsteps = B // (bb * ns)
    scale = -_LN2 / ((C - 1) * B)
    y2 = y.reshape(B, 1).astype(jnp.int32)

    def x_spec(s):
        return pl.BlockSpec((bb, C), lambda i, s=s: (i + s * nsteps, 0))

    out = pl.pallas_call(
        functools.partial(_body, nsteps=nsteps, scale=scale),
        grid=(nsteps,),
        in_specs=[pl.BlockSpec((B, 1), lambda i: (0, 0))]
        + [x_spec(s) for s in range(ns)],
        out_specs=pl.BlockSpec((1, 1), lambda i: (0, 0)),
        out_shape=jax.ShapeDtypeStruct((1, 1), jnp.float32),
        scratch_shapes=[pltpu.VMEM((8, C), jnp.float32)],
    )(y2, *([y_pred] * ns))
    return out[0, 0]
